# baseline (speedup 1.0000x reference)
import jax
import jax.numpy as jnp
import numpy as np
from jax import lax
from jax.experimental import pallas as pl
from jax.experimental.pallas import tpu as pltpu
from jax.experimental.pallas import tpu_sc as plsc

_B = 4096
_TOTAL = 204800
_VOCAB = 100000
_D = 128
_H = 512
_HH = _H // 2
_NC = 2
_NS = 16
_K = 128
_HW = _HH // 2
_UPT = _B // _NS
_CUPAD = 4224
_VR = 1000
_UB = 512


def _pack_bf16_pair(x):
    def rnd(v):
        b = lax.bitcast_convert_type(v, jnp.int32)
        return b + jnp.int32(0x7FFF) + lax.bitwise_and(
            lax.shift_right_logical(b, 16), jnp.int32(1))

    w = lax.bitwise_or(
        lax.shift_right_logical(rnd(x[:, :_HW]), 16),
        lax.bitwise_and(rnd(x[:, _HW:]), jnp.int32(-65536)))
    return lax.bitcast_convert_type(w, jnp.float32)


def _mlp1_body(tab_ref, w1_ref, b1_ref, outa_ref, outb_ref):
    h = jnp.dot(tab_ref[...], w1_ref[...], preferred_element_type=jnp.float32)
    h = jnp.maximum(h + b1_ref[...], 0.0)
    outa_ref[...] = _pack_bf16_pair(h[:, :_HH])
    outb_ref[...] = _pack_bf16_pair(h[:, _HH:])


def _mlp1(table, w1, b1_2d):
    return pl.pallas_call(
        _mlp1_body,
        grid=(_VOCAB // _VR,),
        in_specs=[
            pl.BlockSpec((_VR, _D), lambda i: (i, 0)),
            pl.BlockSpec((_D, _H), lambda i: (0, 0)),
            pl.BlockSpec((1, _H), lambda i: (0, 0)),
        ],
        out_specs=[
            pl.BlockSpec((_VR, _HW), lambda i: (i, 0)),
            pl.BlockSpec((_VR, _HW), lambda i: (i, 0)),
        ],
        out_shape=[
            jax.ShapeDtypeStruct((_VOCAB, _HW), jnp.float32),
            jax.ShapeDtypeStruct((_VOCAB, _HW), jnp.float32),
        ],
    )(table, w1, b1_2d)


def _scal(ref, i):
    return ref[pl.ds(i, 16)][0]


def _chunk_loop(s, ids_hbm, cu_v, t1_hbm, ids0, ids1, rows0, rows1, acc_v,
                sem0, sem1):
    u0 = s * _UPT
    t0 = _scal(cu_v, u0)
    t1 = _scal(cu_v, u0 + _UPT)
    t0a = (t0 // 8) * 8
    nchunks = (t1 - t0a + _K - 1) // _K
    npairs = (nchunks + 1) // 2
    nreg = _HH // 16

    def issue(base, ids_b, rows_b, sem_b):
        pltpu.sync_copy(ids_hbm.at[pl.ds(base, _K)], ids_b)
        pltpu.async_copy(t1_hbm.at[ids_b], rows_b, sem_b)

    def walk(u_in, base, rows_b):
        lo = jnp.maximum(t0, base)
        hi = jnp.minimum(t1, base + _K)

        def ubody(st):
            u, t = st

            def sc(st2):
                return st2[1] <= t

            def sb(st2):
                u2 = st2[0] + 1
                return (u2, _scal(cu_v, u2 + 1))

            u, e_user = lax.while_loop(sc, sb, (u, _scal(cu_v, u + 1)))
            e = jnp.minimum(e_user, hi)
            tl0 = t - base

            def tok(k, regs):
                tl = tl0 + k
                new = list(regs)
                for g in range(8):
                    w = plsc.bitcast(rows_b[tl, pl.ds(g * 16, 16)],
                                     jnp.int32)
                    lo16 = plsc.bitcast(lax.shift_left(w, 16), jnp.float32)
                    hi16 = plsc.bitcast(
                        lax.bitwise_and(w, jnp.int32(-65536)), jnp.float32)
                    new[g] = regs[g] + lo16
                    new[8 + g] = regs[8 + g] + hi16
                return tuple(new)

            regs = plsc.parallel_loop(
                jnp.int32(0), e - t, jnp.int32(1), unroll=4,
                carry=tuple(jnp.zeros((16,), jnp.float32)
                            for _ in range(nreg)))(tok)
            lu = u - u0
            for c0 in range(nreg):
                plsc.addupdate(acc_v.at[lu, pl.ds(c0 * 16, 16)], regs[c0])
            return (u, e)

        u_out, _ = lax.while_loop(lambda st: st[1] < hi, ubody, (u_in, lo))
        return u_out

    issue(t0a, ids0, rows0, sem0)

    def pair(p, u):
        b0 = t0a + (2 * p) * _K
        issue(b0 + _K, ids1, rows1, sem1)
        pltpu.make_async_copy(t1_hbm.at[ids0], rows0, sem0).wait()
        u = walk(u, b0, rows0)
        issue(b0 + 2 * _K, ids0, rows0, sem0)
        pltpu.make_async_copy(t1_hbm.at[ids1], rows1, sem1).wait()
        u = walk(u, b0 + _K, rows1)
        return u

    u_fin = lax.fori_loop(0, npairs, pair, u0, unroll=False)
    pltpu.make_async_copy(t1_hbm.at[ids0], rows0, sem0).wait()
    return u_fin


def _seg_body(ids_hbm, cu_hbm, t1a_hbm, t1b_hbm, outa_hbm, outb_hbm,
              cu_v, ids0, ids1, rows0, rows1, acc_v, sem0, sem1):
    c = lax.axis_index("c")
    s = lax.axis_index("s")
    row0 = s * _UPT

    pltpu.sync_copy(cu_hbm, cu_v)

    def zrow(r, cc):
        for c0 in range(_HH // 16):
            acc_v[r, pl.ds(c0 * 16, 16)] = jnp.zeros((16,), jnp.float32)
        return cc

    lax.fori_loop(0, _UPT, zrow, 0, unroll=False)

    @pl.when(c == 0)
    def _():
        _chunk_loop(s, ids_hbm, cu_v, t1a_hbm, ids0, ids1, rows0, rows1,
                    acc_v, sem0, sem1)
        pltpu.sync_copy(acc_v, outa_hbm.at[pl.ds(row0, _UPT)])

    @pl.when(c == 1)
    def _():
        _chunk_loop(s, ids_hbm, cu_v, t1b_hbm, ids0, ids1, rows0, rows1,
                    acc_v, sem0, sem1)
        pltpu.sync_copy(acc_v, outb_hbm.at[pl.ds(row0, _UPT)])


def _segsum(ids_padded, cu_pad, t1a, t1b):
    return pl.kernel(
        _seg_body,
        out_type=(
            jax.ShapeDtypeStruct((_B, _HH), jnp.float32),
            jax.ShapeDtypeStruct((_B, _HH), jnp.float32),
        ),
        mesh=plsc.VectorSubcoreMesh(
            core_axis_name="c", subcore_axis_name="s",
            num_cores=_NC, num_subcores=_NS,
        ),
        scratch_types=[
            pltpu.VMEM((_CUPAD,), jnp.int32),
            pltpu.VMEM((_K,), jnp.int32),
            pltpu.VMEM((_K,), jnp.int32),
            pltpu.VMEM((_K, _HW), jnp.float32),
            pltpu.VMEM((_K, _HW), jnp.float32),
            pltpu.VMEM((_UPT, _HH), jnp.float32),
            pltpu.SemaphoreType.DMA,
            pltpu.SemaphoreType.DMA,
        ],
        compiler_params=pltpu.CompilerParams(needs_layout_passes=False),
    )(ids_padded, cu_pad, t1a, t1b)


def _out_body(a_ref, b_ref, lo_ref, hi_ref, w2a_ref, w2b_ref, b2_ref, o_ref):
    cnt = (hi_ref[...] - lo_ref[...]).astype(jnp.float32)
    inv = 1.0 / jnp.maximum(cnt, 1.0)
    y = jnp.dot(a_ref[...] * inv, w2a_ref[...],
                preferred_element_type=jnp.float32)
    y = y + jnp.dot(b_ref[...] * inv, w2b_ref[...],
                    preferred_element_type=jnp.float32)
    y = y + b2_ref[...]
    o_ref[...] = jnp.where(cnt > 0.0, y, jnp.zeros_like(y))


def _finish(suma, sumb, cu_lo, cu_hi, w2a, w2b, b2_2d):
    return pl.pallas_call(
        _out_body,
        grid=(_B // _UB,),
        in_specs=[
            pl.BlockSpec((_UB, _HH), lambda i: (i, 0)),
            pl.BlockSpec((_UB, _HH), lambda i: (i, 0)),
            pl.BlockSpec((_UB, 1), lambda i: (i, 0)),
            pl.BlockSpec((_UB, 1), lambda i: (i, 0)),
            pl.BlockSpec((_HH, _D), lambda i: (0, 0)),
            pl.BlockSpec((_HH, _D), lambda i: (0, 0)),
            pl.BlockSpec((1, _D), lambda i: (0, 0)),
        ],
        out_specs=pl.BlockSpec((_UB, _D), lambda i: (i, 0)),
        out_shape=jax.ShapeDtypeStruct((_B, _D), jnp.float32),
    )(suma, sumb, cu_lo, cu_hi, w2a, w2b, b2_2d)


def kernel(flat_movie_ids, cu_seqlens, table, W1, b1, W2, b2):
    t1a, t1b = _mlp1(table, W1, b1.reshape(1, _H))
    cu_pad = jnp.concatenate([
        cu_seqlens,
        jnp.full((_CUPAD - _B - 1,), jnp.int32(0x3FFFFFFF), jnp.int32),
    ])
    ids_padded = jnp.concatenate([
        flat_movie_ids, jnp.zeros((3 * _K,), jnp.int32),
    ])
    suma, sumb = _segsum(ids_padded, cu_pad, t1a, t1b)
    out = _finish(
        suma, sumb,
        cu_seqlens[:-1].reshape(_B, 1), cu_seqlens[1:].reshape(_B, 1),
        W2[:_HH], W2[_HH:], b2.reshape(1, _D),
    )
    return out


# speedup vs baseline: 1.0797x; 1.0797x over previous
import jax
import jax.numpy as jnp
import numpy as np
from jax import lax
from jax.experimental import pallas as pl
from jax.experimental.pallas import tpu as pltpu
from jax.experimental.pallas import tpu_sc as plsc

_B = 4096
_TOTAL = 204800
_VOCAB = 100000
_D = 128
_H = 512
_HH = _H // 2
_NC = 2
_NS = 16
_K = 128
_HW = _HH // 2
_UPT = _B // _NS
_CUPAD = 4224
_VR = 1000
_UB = 512


def _pack_bf16_pair(x):
    def rnd(v):
        b = lax.bitcast_convert_type(v, jnp.int32)
        return b + jnp.int32(0x8000)

    w = lax.bitwise_or(
        lax.shift_right_logical(rnd(x[:, :_HW]), 16),
        lax.bitwise_and(rnd(x[:, _HW:]), jnp.int32(-65536)))
    return lax.bitcast_convert_type(w, jnp.float32)


def _mlp1_body(tab_ref, w1_ref, b1_ref, outa_ref, outb_ref):
    h = jnp.dot(tab_ref[...], w1_ref[...], preferred_element_type=jnp.float32)
    h = jnp.maximum(h + b1_ref[...], 0.0)
    outa_ref[...] = _pack_bf16_pair(h[:, :_HH])
    outb_ref[...] = _pack_bf16_pair(h[:, _HH:])


def _mlp1(table, w1, b1_2d):
    return pl.pallas_call(
        _mlp1_body,
        grid=(_VOCAB // _VR,),
        in_specs=[
            pl.BlockSpec((_VR, _D), lambda i: (i, 0)),
            pl.BlockSpec((_D, _H), lambda i: (0, 0)),
            pl.BlockSpec((1, _H), lambda i: (0, 0)),
        ],
        out_specs=[
            pl.BlockSpec((_VR, _HW), lambda i: (i, 0)),
            pl.BlockSpec((_VR, _HW), lambda i: (i, 0)),
        ],
        out_shape=[
            jax.ShapeDtypeStruct((_VOCAB, _HW), jnp.float32),
            jax.ShapeDtypeStruct((_VOCAB, _HW), jnp.float32),
        ],
    )(table, w1, b1_2d)


def _scal(ref, i):
    return ref[pl.ds(i, 16)][0]


def _chunk_loop(s, ids_hbm, cu_v, t1_hbm, ids0, ids1, rows0, rows1, acc_v,
                sem0, sem1):
    u0 = s * _UPT
    t0 = _scal(cu_v, u0)
    t1 = _scal(cu_v, u0 + _UPT)
    t0a = (t0 // 8) * 8
    nchunks = (t1 - t0a + _K - 1) // _K
    npairs = (nchunks + 1) // 2
    nreg = _HH // 16

    def issue(base, ids_b, rows_b, sem_b):
        pltpu.sync_copy(ids_hbm.at[pl.ds(base, _K)], ids_b)
        pltpu.async_copy(t1_hbm.at[ids_b], rows_b, sem_b)

    def walk(st_in, base, rows_b):
        hi = jnp.minimum(t1, base + _K)

        def ubody(st):
            u, t = st

            def sc(st2):
                return st2[1] <= t

            def sb(st2):
                u2 = st2[0] + 1
                return (u2, _scal(cu_v, u2 + 1))

            u, e_user = lax.while_loop(sc, sb, (u, _scal(cu_v, u + 1)))
            e = jnp.minimum(e_user, hi)
            tl0 = t - base

            def tok(k, regs):
                tl = tl0 + k
                new = list(regs)
                for g in range(8):
                    w = plsc.bitcast(rows_b[tl, pl.ds(g * 16, 16)],
                                     jnp.int32)
                    lo16 = plsc.bitcast(lax.shift_left(w, 16), jnp.float32)
                    hi16 = plsc.bitcast(
                        lax.bitwise_and(w, jnp.int32(-65536)), jnp.float32)
                    new[g] = regs[g] + lo16
                    new[8 + g] = regs[8 + g] + hi16
                return tuple(new)

            regs = lax.fori_loop(
                0, e - t, tok,
                tuple(jnp.zeros((16,), jnp.float32) for _ in range(nreg)),
                unroll=False)
            lu = u - u0
            for c0 in range(nreg):
                plsc.addupdate(acc_v.at[lu, pl.ds(c0 * 16, 16)], regs[c0])
            return (u, e)

        return lax.while_loop(lambda st: st[1] < hi, ubody, st_in)

    def cbase(i):
        return jnp.minimum(t0a + i * _K, _TOTAL - _K)

    issue(cbase(0), ids0, rows0, sem0)

    def pair(p, st):
        i0 = 2 * p
        issue(cbase(i0 + 1), ids1, rows1, sem1)
        pltpu.make_async_copy(t1_hbm.at[ids0], rows0, sem0).wait()
        st = walk(st, cbase(i0), rows0)
        issue(cbase(i0 + 2), ids0, rows0, sem0)
        pltpu.make_async_copy(t1_hbm.at[ids1], rows1, sem1).wait()
        st = walk(st, cbase(i0 + 1), rows1)
        return st

    st_fin = lax.fori_loop(0, npairs, pair, (u0, t0), unroll=False)
    pltpu.make_async_copy(t1_hbm.at[ids0], rows0, sem0).wait()
    return st_fin


def _seg_body(ids_hbm, cu_hbm, t1a_hbm, t1b_hbm, outa_hbm, outb_hbm,
              cu_v, ids0, ids1, rows0, rows1, acc_v, sem0, sem1):
    c = lax.axis_index("c")
    s = lax.axis_index("s")
    row0 = s * _UPT

    pltpu.sync_copy(cu_hbm, cu_v)

    def zrow(r, cc):
        for c0 in range(_HH // 16):
            acc_v[r, pl.ds(c0 * 16, 16)] = jnp.zeros((16,), jnp.float32)
        return cc

    lax.fori_loop(0, _UPT, zrow, 0, unroll=False)

    @pl.when(c == 0)
    def _():
        _chunk_loop(s, ids_hbm, cu_v, t1a_hbm, ids0, ids1, rows0, rows1,
                    acc_v, sem0, sem1)
        pltpu.sync_copy(acc_v, outa_hbm.at[pl.ds(row0, _UPT)])

    @pl.when(c == 1)
    def _():
        _chunk_loop(s, ids_hbm, cu_v, t1b_hbm, ids0, ids1, rows0, rows1,
                    acc_v, sem0, sem1)
        pltpu.sync_copy(acc_v, outb_hbm.at[pl.ds(row0, _UPT)])


def _segsum(ids_padded, cu_pad, t1a, t1b):
    return pl.kernel(
        _seg_body,
        out_type=(
            jax.ShapeDtypeStruct((_B, _HH), jnp.float32),
            jax.ShapeDtypeStruct((_B, _HH), jnp.float32),
        ),
        mesh=plsc.VectorSubcoreMesh(
            core_axis_name="c", subcore_axis_name="s",
            num_cores=_NC, num_subcores=_NS,
        ),
        scratch_types=[
            pltpu.VMEM((_CUPAD,), jnp.int32),
            pltpu.VMEM((_K,), jnp.int32),
            pltpu.VMEM((_K,), jnp.int32),
            pltpu.VMEM((_K, _HW), jnp.float32),
            pltpu.VMEM((_K, _HW), jnp.float32),
            pltpu.VMEM((_UPT, _HH), jnp.float32),
            pltpu.SemaphoreType.DMA,
            pltpu.SemaphoreType.DMA,
        ],
        compiler_params=pltpu.CompilerParams(needs_layout_passes=False),
    )(ids_padded, cu_pad, t1a, t1b)


def _out_body(a_ref, b_ref, lo_ref, hi_ref, w2a_ref, w2b_ref, b2_ref, o_ref):
    cnt = (hi_ref[...] - lo_ref[...]).astype(jnp.float32)
    inv = 1.0 / jnp.maximum(cnt, 1.0)
    y = jnp.dot(a_ref[...] * inv, w2a_ref[...],
                preferred_element_type=jnp.float32)
    y = y + jnp.dot(b_ref[...] * inv, w2b_ref[...],
                    preferred_element_type=jnp.float32)
    y = y + b2_ref[...]
    o_ref[...] = jnp.where(cnt > 0.0, y, jnp.zeros_like(y))


def _finish(suma, sumb, cu_lo, cu_hi, w2a, w2b, b2_2d):
    return pl.pallas_call(
        _out_body,
        grid=(_B // _UB,),
        in_specs=[
            pl.BlockSpec((_UB, _HH), lambda i: (i, 0)),
            pl.BlockSpec((_UB, _HH), lambda i: (i, 0)),
            pl.BlockSpec((_UB, 1), lambda i: (i, 0)),
            pl.BlockSpec((_UB, 1), lambda i: (i, 0)),
            pl.BlockSpec((_HH, _D), lambda i: (0, 0)),
            pl.BlockSpec((_HH, _D), lambda i: (0, 0)),
            pl.BlockSpec((1, _D), lambda i: (0, 0)),
        ],
        out_specs=pl.BlockSpec((_UB, _D), lambda i: (i, 0)),
        out_shape=jax.ShapeDtypeStruct((_B, _D), jnp.float32),
    )(suma, sumb, cu_lo, cu_hi, w2a, w2b, b2_2d)


def kernel(flat_movie_ids, cu_seqlens, table, W1, b1, W2, b2):
    t1a, t1b = _mlp1(table, W1, b1.reshape(1, _H))
    cu_pad = jnp.concatenate([
        cu_seqlens,
        jnp.full((_CUPAD - _B - 1,), jnp.int32(0x3FFFFFFF), jnp.int32),
    ])
    suma, sumb = _segsum(flat_movie_ids, cu_pad, t1a, t1b)
    out = _finish(
        suma, sumb,
        cu_seqlens[:-1].reshape(_B, 1), cu_seqlens[1:].reshape(_B, 1),
        W2[:_HH], W2[_HH:], b2.reshape(1, _D),
    )
    return out


# speedup vs baseline: 1.2012x; 1.1125x over previous
import jax
import jax.numpy as jnp
import numpy as np
from jax import lax
from jax.experimental import pallas as pl
from jax.experimental.pallas import tpu as pltpu
from jax.experimental.pallas import tpu_sc as plsc

_B = 4096
_TOTAL = 204800
_VOCAB = 100000
_D = 128
_H = 512
_HH = _H // 2
_NC = 2
_NS = 16
_K = 128
_HW = _HH // 2
_UPT = _B // _NS
_CUPAD = 4224
_VR = 1000
_UB = 512


def _pack_bf16_pair(x):
    def rnd(v):
        b = lax.bitcast_convert_type(v, jnp.int32)
        return b + jnp.int32(0x8000)

    w = lax.bitwise_or(
        lax.shift_right_logical(rnd(x[:, :_HW]), 16),
        lax.bitwise_and(rnd(x[:, _HW:]), jnp.int32(-65536)))
    return lax.bitcast_convert_type(w, jnp.float32)


def _mlp1_body(tab_ref, w1_ref, b1_ref, outa_ref, outb_ref):
    h = jnp.dot(tab_ref[...], w1_ref[...], preferred_element_type=jnp.float32)
    h = jnp.maximum(h + b1_ref[...], 0.0)
    outa_ref[...] = _pack_bf16_pair(h[:, :_HH])
    outb_ref[...] = _pack_bf16_pair(h[:, _HH:])


def _mlp1(table, w1, b1_2d):
    return pl.pallas_call(
        _mlp1_body,
        grid=(_VOCAB // _VR,),
        in_specs=[
            pl.BlockSpec((_VR, _D), lambda i: (i, 0)),
            pl.BlockSpec((_D, _H), lambda i: (0, 0)),
            pl.BlockSpec((1, _H), lambda i: (0, 0)),
        ],
        out_specs=[
            pl.BlockSpec((_VR, _HW), lambda i: (i, 0)),
            pl.BlockSpec((_VR, _HW), lambda i: (i, 0)),
        ],
        out_shape=[
            jax.ShapeDtypeStruct((_VOCAB, _HW), jnp.float32),
            jax.ShapeDtypeStruct((_VOCAB, _HW), jnp.float32),
        ],
    )(table, w1, b1_2d)


def _scal(ref, i):
    return ref[pl.ds(i, 16)][0]


def _chunk_loop(s, ids_hbm, cu_v, t1_hbm, ids0, ids1, rows0, rows1, acc_v,
                gsem0, gsem1, isem0, isem1):
    u0 = s * _UPT
    t0 = _scal(cu_v, u0)
    t1 = _scal(cu_v, u0 + _UPT)
    t0a = (t0 // 8) * 8
    nchunks = (t1 - t0a + _K - 1) // _K
    npairs = (nchunks + 1) // 2
    nreg = _HH // 16

    def walk(st_in, base, rows_b):
        hi = jnp.minimum(t1, base + _K)

        def ubody(st):
            u, t = st

            def sc(st2):
                return st2[1] <= t

            def sb(st2):
                u2 = st2[0] + 1
                return (u2, _scal(cu_v, u2 + 1))

            u, e_user = lax.while_loop(sc, sb, (u, _scal(cu_v, u + 1)))
            e = jnp.minimum(e_user, hi)
            tl0 = t - base

            def tok(k, regs):
                tl = tl0 + k
                new = list(regs)
                for g in range(8):
                    w = plsc.bitcast(rows_b[tl, pl.ds(g * 16, 16)],
                                     jnp.int32)
                    lo16 = plsc.bitcast(lax.shift_left(w, 16), jnp.float32)
                    hi16 = plsc.bitcast(
                        lax.bitwise_and(w, jnp.int32(-65536)), jnp.float32)
                    new[g] = regs[g] + lo16
                    new[8 + g] = regs[8 + g] + hi16
                return tuple(new)

            regs = lax.fori_loop(
                0, e - t, tok,
                tuple(jnp.zeros((16,), jnp.float32) for _ in range(nreg)),
                unroll=False)
            lu = u - u0
            for c0 in range(nreg):
                plsc.addupdate(acc_v.at[lu, pl.ds(c0 * 16, 16)], regs[c0])
            return (u, e)

        return lax.while_loop(lambda st: st[1] < hi, ubody, st_in)

    def cbase(i):
        return jnp.minimum(t0a + i * _K, _TOTAL - _K)

    def ids_start(i, ids_b, isem_b):
        pltpu.async_copy(ids_hbm.at[pl.ds(cbase(i), _K)], ids_b, isem_b)

    def ids_wait(i, ids_b, isem_b):
        pltpu.make_async_copy(
            ids_hbm.at[pl.ds(cbase(i), _K)], ids_b, isem_b).wait()

    pltpu.sync_copy(ids_hbm.at[pl.ds(cbase(0), _K)], ids0)
    pltpu.async_copy(t1_hbm.at[ids0], rows0, gsem0)
    ids_start(1, ids1, isem1)

    def pair(p, st):
        ia = 2 * p + 1
        ids_wait(ia, ids1, isem1)
        pltpu.async_copy(t1_hbm.at[ids1], rows1, gsem1)
        pltpu.make_async_copy(t1_hbm.at[ids0], rows0, gsem0).wait()
        ids_start(ia + 1, ids0, isem0)
        st = walk(st, cbase(ia - 1), rows0)
        ids_wait(ia + 1, ids0, isem0)
        pltpu.async_copy(t1_hbm.at[ids0], rows0, gsem0)
        pltpu.make_async_copy(t1_hbm.at[ids1], rows1, gsem1).wait()
        ids_start(ia + 2, ids1, isem1)
        st = walk(st, cbase(ia), rows1)
        return st

    st_fin = lax.fori_loop(0, npairs, pair, (u0, t0), unroll=False)
    pltpu.make_async_copy(t1_hbm.at[ids0], rows0, gsem0).wait()
    ids_wait(2 * npairs + 1, ids1, isem1)
    return st_fin


def _seg_body(ids_hbm, cu_hbm, t1a_hbm, t1b_hbm, outa_hbm, outb_hbm,
              cu_v, ids0, ids1, rows0, rows1, acc_v,
              gsem0, gsem1, isem0, isem1):
    c = lax.axis_index("c")
    s = lax.axis_index("s")
    row0 = s * _UPT

    pltpu.sync_copy(cu_hbm, cu_v)

    def zrow(r, cc):
        for c0 in range(_HH // 16):
            acc_v[r, pl.ds(c0 * 16, 16)] = jnp.zeros((16,), jnp.float32)
        return cc

    lax.fori_loop(0, _UPT, zrow, 0, unroll=False)

    @pl.when(c == 0)
    def _():
        _chunk_loop(s, ids_hbm, cu_v, t1a_hbm, ids0, ids1, rows0, rows1,
                    acc_v, gsem0, gsem1, isem0, isem1)
        pltpu.sync_copy(acc_v, outa_hbm.at[pl.ds(row0, _UPT)])

    @pl.when(c == 1)
    def _():
        _chunk_loop(s, ids_hbm, cu_v, t1b_hbm, ids0, ids1, rows0, rows1,
                    acc_v, gsem0, gsem1, isem0, isem1)
        pltpu.sync_copy(acc_v, outb_hbm.at[pl.ds(row0, _UPT)])


def _segsum(ids_padded, cu_pad, t1a, t1b):
    return pl.kernel(
        _seg_body,
        out_type=(
            jax.ShapeDtypeStruct((_B, _HH), jnp.float32),
            jax.ShapeDtypeStruct((_B, _HH), jnp.float32),
        ),
        mesh=plsc.VectorSubcoreMesh(
            core_axis_name="c", subcore_axis_name="s",
            num_cores=_NC, num_subcores=_NS,
        ),
        scratch_types=[
            pltpu.VMEM((_CUPAD,), jnp.int32),
            pltpu.VMEM((_K,), jnp.int32),
            pltpu.VMEM((_K,), jnp.int32),
            pltpu.VMEM((_K, _HW), jnp.float32),
            pltpu.VMEM((_K, _HW), jnp.float32),
            pltpu.VMEM((_UPT, _HH), jnp.float32),
            pltpu.SemaphoreType.DMA,
            pltpu.SemaphoreType.DMA,
            pltpu.SemaphoreType.DMA,
            pltpu.SemaphoreType.DMA,
        ],
        compiler_params=pltpu.CompilerParams(needs_layout_passes=False),
    )(ids_padded, cu_pad, t1a, t1b)


def _out_body(a_ref, b_ref, lo_ref, hi_ref, w2a_ref, w2b_ref, b2_ref, o_ref):
    cnt = (hi_ref[...] - lo_ref[...]).astype(jnp.float32)
    inv = 1.0 / jnp.maximum(cnt, 1.0)
    y = jnp.dot(a_ref[...] * inv, w2a_ref[...],
                preferred_element_type=jnp.float32)
    y = y + jnp.dot(b_ref[...] * inv, w2b_ref[...],
                    preferred_element_type=jnp.float32)
    y = y + b2_ref[...]
    o_ref[...] = jnp.where(cnt > 0.0, y, jnp.zeros_like(y))


def _finish(suma, sumb, cu_lo, cu_hi, w2a, w2b, b2_2d):
    return pl.pallas_call(
        _out_body,
        grid=(_B // _UB,),
        in_specs=[
            pl.BlockSpec((_UB, _HH), lambda i: (i, 0)),
            pl.BlockSpec((_UB, _HH), lambda i: (i, 0)),
            pl.BlockSpec((_UB, 1), lambda i: (i, 0)),
            pl.BlockSpec((_UB, 1), lambda i: (i, 0)),
            pl.BlockSpec((_HH, _D), lambda i: (0, 0)),
            pl.BlockSpec((_HH, _D), lambda i: (0, 0)),
            pl.BlockSpec((1, _D), lambda i: (0, 0)),
        ],
        out_specs=pl.BlockSpec((_UB, _D), lambda i: (i, 0)),
        out_shape=jax.ShapeDtypeStruct((_B, _D), jnp.float32),
    )(suma, sumb, cu_lo, cu_hi, w2a, w2b, b2_2d)


def kernel(flat_movie_ids, cu_seqlens, table, W1, b1, W2, b2):
    t1a, t1b = _mlp1(table, W1, b1.reshape(1, _H))
    cu_pad = jnp.concatenate([
        cu_seqlens,
        jnp.full((_CUPAD - _B - 1,), jnp.int32(0x3FFFFFFF), jnp.int32),
    ])
    suma, sumb = _segsum(flat_movie_ids, cu_pad, t1a, t1b)
    out = _finish(
        suma, sumb,
        cu_seqlens[:-1].reshape(_B, 1), cu_seqlens[1:].reshape(_B, 1),
        W2[:_HH], W2[_HH:], b2.reshape(1, _D),
    )
    return out


# speedup vs baseline: 1.3973x; 1.1633x over previous
import jax
import jax.numpy as jnp
import numpy as np
from jax import lax
from jax.experimental import pallas as pl
from jax.experimental.pallas import tpu as pltpu
from jax.experimental.pallas import tpu_sc as plsc

_B = 4096
_TOTAL = 204800
_VOCAB = 100000
_D = 128
_H = 512
_HH = _H // 2
_NC = 2
_NS = 16
_K = 128
_HW = _HH // 2
_UPT = _B // _NS
_CUPAD = 4224
_VR = 2000
_UB = 512


def _pack_bf16_pair(x):
    def rnd(v):
        b = lax.bitcast_convert_type(v, jnp.int32)
        return b + jnp.int32(0x8000)

    w = lax.bitwise_or(
        lax.shift_right_logical(rnd(x[:, :_HW]), 16),
        lax.bitwise_and(rnd(x[:, _HW:]), jnp.int32(-65536)))
    return lax.bitcast_convert_type(w, jnp.float32)


def _mlp1_body(tab_ref, w1_ref, b1_ref, outa_ref, outb_ref):
    h = jnp.dot(tab_ref[...], w1_ref[...], preferred_element_type=jnp.float32)
    h = jnp.maximum(h + b1_ref[...], 0.0)
    outa_ref[...] = _pack_bf16_pair(h[:, :_HH])
    outb_ref[...] = _pack_bf16_pair(h[:, _HH:])


def _mlp1(table, w1, b1_2d):
    return pl.pallas_call(
        _mlp1_body,
        grid=(_VOCAB // _VR,),
        in_specs=[
            pl.BlockSpec((_VR, _D), lambda i: (i, 0)),
            pl.BlockSpec((_D, _H), lambda i: (0, 0)),
            pl.BlockSpec((1, _H), lambda i: (0, 0)),
        ],
        out_specs=[
            pl.BlockSpec((_VR, _HW), lambda i: (i, 0)),
            pl.BlockSpec((_VR, _HW), lambda i: (i, 0)),
        ],
        out_shape=[
            jax.ShapeDtypeStruct((_VOCAB, _HW), jnp.float32),
            jax.ShapeDtypeStruct((_VOCAB, _HW), jnp.float32),
        ],
    )(table, w1, b1_2d)


def _scal(ref, i):
    return ref[pl.ds(i, 16)][0]


def _chunk_loop(s, ids_hbm, cu_v, t1_hbm, ids0, ids1, rows0, rows1, acc_v,
                gsem0, gsem1, isem0, isem1):
    u0 = s * _UPT
    t0 = _scal(cu_v, u0)
    t1 = _scal(cu_v, u0 + _UPT)
    t0a = (t0 // 8) * 8
    nchunks = (t1 - t0a + _K - 1) // _K
    npairs = (nchunks + 1) // 2
    nreg = _HH // 16

    def walk(st_in, base, rows_b):
        hi = jnp.minimum(t1, base + _K)

        def ubody(st):
            u, t = st

            def sc(st2):
                return st2[1] <= t

            def sb(st2):
                u2 = st2[0] + 1
                return (u2, _scal(cu_v, u2 + 1))

            u, e_user = lax.while_loop(sc, sb, (u, _scal(cu_v, u + 1)))
            e = jnp.minimum(e_user, hi)
            tl0 = t - base

            def tok(k, regs):
                tl = tl0 + k
                new = list(regs)
                for g in range(8):
                    w = plsc.bitcast(rows_b[tl, pl.ds(g * 16, 16)],
                                     jnp.int32)
                    lo16 = plsc.bitcast(lax.shift_left(w, 16), jnp.float32)
                    hi16 = plsc.bitcast(w, jnp.float32)
                    new[g] = regs[g] + lo16
                    new[8 + g] = regs[8 + g] + hi16
                return tuple(new)

            regs = plsc.parallel_loop(
                jnp.int32(0), e - t, jnp.int32(1), unroll=2,
                carry=tuple(jnp.zeros((16,), jnp.float32)
                            for _ in range(nreg)))(tok)
            lu = u - u0
            for c0 in range(nreg):
                plsc.addupdate(acc_v.at[lu, pl.ds(c0 * 16, 16)], regs[c0])
            return (u, e)

        return lax.while_loop(lambda st: st[1] < hi, ubody, st_in)

    def cbase(i):
        return jnp.minimum(t0a + i * _K, _TOTAL - _K)

    def ids_start(i, ids_b, isem_b):
        pltpu.async_copy(ids_hbm.at[pl.ds(cbase(i), _K)], ids_b, isem_b)

    def ids_wait(i, ids_b, isem_b):
        pltpu.make_async_copy(
            ids_hbm.at[pl.ds(cbase(i), _K)], ids_b, isem_b).wait()

    pltpu.sync_copy(ids_hbm.at[pl.ds(cbase(0), _K)], ids0)
    pltpu.async_copy(t1_hbm.at[ids0], rows0, gsem0)
    ids_start(1, ids1, isem1)

    def pair(p, st):
        ia = 2 * p + 1
        ids_wait(ia, ids1, isem1)
        pltpu.async_copy(t1_hbm.at[ids1], rows1, gsem1)
        pltpu.make_async_copy(t1_hbm.at[ids0], rows0, gsem0).wait()
        ids_start(ia + 1, ids0, isem0)
        st = walk(st, cbase(ia - 1), rows0)
        ids_wait(ia + 1, ids0, isem0)
        pltpu.async_copy(t1_hbm.at[ids0], rows0, gsem0)
        pltpu.make_async_copy(t1_hbm.at[ids1], rows1, gsem1).wait()
        ids_start(ia + 2, ids1, isem1)
        st = walk(st, cbase(ia), rows1)
        return st

    st_fin = lax.fori_loop(0, npairs, pair, (u0, t0), unroll=False)
    pltpu.make_async_copy(t1_hbm.at[ids0], rows0, gsem0).wait()
    ids_wait(2 * npairs + 1, ids1, isem1)
    return st_fin


def _seg_body(ids_hbm, cu_hbm, t1a_hbm, t1b_hbm, outa_hbm, outb_hbm,
              cu_v, ids0, ids1, rows0, rows1, acc_v,
              gsem0, gsem1, isem0, isem1):
    c = lax.axis_index("c")
    s = lax.axis_index("s")
    row0 = s * _UPT

    pltpu.sync_copy(cu_hbm, cu_v.at[pl.ds(0, _B + 1)])

    def zrow(r, cc):
        for c0 in range(_HH // 16):
            acc_v[r, pl.ds(c0 * 16, 16)] = jnp.zeros((16,), jnp.float32)
        return cc

    lax.fori_loop(0, _UPT, zrow, 0, unroll=False)

    @pl.when(c == 0)
    def _():
        _chunk_loop(s, ids_hbm, cu_v, t1a_hbm, ids0, ids1, rows0, rows1,
                    acc_v, gsem0, gsem1, isem0, isem1)
        pltpu.sync_copy(acc_v, outa_hbm.at[pl.ds(row0, _UPT)])

    @pl.when(c == 1)
    def _():
        _chunk_loop(s, ids_hbm, cu_v, t1b_hbm, ids0, ids1, rows0, rows1,
                    acc_v, gsem0, gsem1, isem0, isem1)
        pltpu.sync_copy(acc_v, outb_hbm.at[pl.ds(row0, _UPT)])


def _segsum(ids_padded, cu_pad, t1a, t1b):
    return pl.kernel(
        _seg_body,
        out_type=(
            jax.ShapeDtypeStruct((_B, _HH), jnp.float32),
            jax.ShapeDtypeStruct((_B, _HH), jnp.float32),
        ),
        mesh=plsc.VectorSubcoreMesh(
            core_axis_name="c", subcore_axis_name="s",
            num_cores=_NC, num_subcores=_NS,
        ),
        scratch_types=[
            pltpu.VMEM((_CUPAD,), jnp.int32),
            pltpu.VMEM((_K,), jnp.int32),
            pltpu.VMEM((_K,), jnp.int32),
            pltpu.VMEM((_K, _HW), jnp.float32),
            pltpu.VMEM((_K, _HW), jnp.float32),
            pltpu.VMEM((_UPT, _HH), jnp.float32),
            pltpu.SemaphoreType.DMA,
            pltpu.SemaphoreType.DMA,
            pltpu.SemaphoreType.DMA,
            pltpu.SemaphoreType.DMA,
        ],
        compiler_params=pltpu.CompilerParams(needs_layout_passes=False),
    )(ids_padded, cu_pad, t1a, t1b)


def _out_body(a_ref, b_ref, lo_ref, hi_ref, w2a_ref, w2b_ref, b2_ref, o_ref):
    cnt = (hi_ref[...] - lo_ref[...]).astype(jnp.float32)
    inv = 1.0 / jnp.maximum(cnt, 1.0)
    y = jnp.dot(a_ref[...] * inv, w2a_ref[...],
                preferred_element_type=jnp.float32)
    y = y + jnp.dot(b_ref[...] * inv, w2b_ref[...],
                    preferred_element_type=jnp.float32)
    y = y + b2_ref[...]
    o_ref[...] = jnp.where(cnt > 0.0, y, jnp.zeros_like(y))


def _finish(suma, sumb, cu_lo, cu_hi, w2a, w2b, b2_2d):
    return pl.pallas_call(
        _out_body,
        grid=(_B // _UB,),
        in_specs=[
            pl.BlockSpec((_UB, _HH), lambda i: (i, 0)),
            pl.BlockSpec((_UB, _HH), lambda i: (i, 0)),
            pl.BlockSpec((_UB, 1), lambda i: (i, 0)),
            pl.BlockSpec((_UB, 1), lambda i: (i, 0)),
            pl.BlockSpec((_HH, _D), lambda i: (0, 0)),
            pl.BlockSpec((_HH, _D), lambda i: (0, 0)),
            pl.BlockSpec((1, _D), lambda i: (0, 0)),
        ],
        out_specs=pl.BlockSpec((_UB, _D), lambda i: (i, 0)),
        out_shape=jax.ShapeDtypeStruct((_B, _D), jnp.float32),
    )(suma, sumb, cu_lo, cu_hi, w2a, w2b, b2_2d)


def kernel(flat_movie_ids, cu_seqlens, table, W1, b1, W2, b2):
    t1a, t1b = _mlp1(table, W1, b1.reshape(1, _H))
    suma, sumb = _segsum(flat_movie_ids, cu_seqlens, t1a, t1b)
    out = _finish(
        suma, sumb,
        cu_seqlens[:-1].reshape(_B, 1), cu_seqlens[1:].reshape(_B, 1),
        W2[:_HH], W2[_HH:], b2.reshape(1, _D),
    )
    return out


# speedup vs baseline: 1.4856x; 1.0632x over previous
import jax
import jax.numpy as jnp
import numpy as np
from jax import lax
from jax.experimental import pallas as pl
from jax.experimental.pallas import tpu as pltpu
from jax.experimental.pallas import tpu_sc as plsc

_B = 4096
_TOTAL = 204800
_VOCAB = 100000
_D = 128
_H = 512
_HH = _H // 2
_NC = 2
_NS = 16
_K = 128
_HW = _HH // 2
_UPT = _B // _NS
_CUPAD = 4224
_VR = 4000
_UB = 512


def _pack_bf16_pair(x):
    def rnd(v):
        b = lax.bitcast_convert_type(v, jnp.int32)
        return b + jnp.int32(0x8000)

    w = lax.bitwise_or(
        lax.shift_right_logical(rnd(x[:, :_HW]), 16),
        lax.bitwise_and(rnd(x[:, _HW:]), jnp.int32(-65536)))
    return lax.bitcast_convert_type(w, jnp.float32)


def _mlp1_body(tab_ref, w1_ref, b1_ref, outa_ref, outb_ref):
    h = jnp.dot(tab_ref[...], w1_ref[...], preferred_element_type=jnp.float32)
    h = jnp.maximum(h + b1_ref[...], 0.0)
    outa_ref[...] = _pack_bf16_pair(h[:, :_HH])
    outb_ref[...] = _pack_bf16_pair(h[:, _HH:])


def _mlp1(table, w1, b1_2d):
    return pl.pallas_call(
        _mlp1_body,
        grid=(_VOCAB // _VR,),
        in_specs=[
            pl.BlockSpec((_VR, _D), lambda i: (i, 0)),
            pl.BlockSpec((_D, _H), lambda i: (0, 0)),
            pl.BlockSpec((1, _H), lambda i: (0, 0)),
        ],
        out_specs=[
            pl.BlockSpec((_VR, _HW), lambda i: (i, 0)),
            pl.BlockSpec((_VR, _HW), lambda i: (i, 0)),
        ],
        out_shape=[
            jax.ShapeDtypeStruct((_VOCAB, _HW), jnp.float32),
            jax.ShapeDtypeStruct((_VOCAB, _HW), jnp.float32),
        ],
    )(table, w1, b1_2d)


def _scal(ref, i):
    return ref[pl.ds(i, 16)][0]


def _chunk_loop(s, ids_hbm, cu_v, t1_hbm, ids0, ids1, rows0, rows1, acc_v,
                gsem0, gsem1, isem0, isem1):
    u0 = s * _UPT
    t0 = _scal(cu_v, u0)
    t1 = _scal(cu_v, u0 + _UPT)
    t0a = (t0 // 8) * 8
    nchunks = (t1 - t0a + _K - 1) // _K
    npairs = (nchunks + 1) // 2
    nreg = _HH // 16

    def walk(st_in, base, rows_b):
        hi = jnp.minimum(t1, base + _K)

        def ubody(st):
            u, t = st

            def sc(st2):
                return st2[1] <= t

            def sb(st2):
                u2 = st2[0] + 1
                return (u2, _scal(cu_v, u2 + 1))

            u, e_user = lax.while_loop(sc, sb, (u, _scal(cu_v, u + 1)))
            e = jnp.minimum(e_user, hi)
            tl0 = t - base

            def tok(k, regs):
                tl = tl0 + k
                new = list(regs)
                for g in range(8):
                    w = plsc.bitcast(rows_b[tl, pl.ds(g * 16, 16)],
                                     jnp.int32)
                    lo16 = plsc.bitcast(lax.shift_left(w, 16), jnp.float32)
                    hi16 = plsc.bitcast(w, jnp.float32)
                    new[g] = regs[g] + lo16
                    new[8 + g] = regs[8 + g] + hi16
                return tuple(new)

            regs = plsc.parallel_loop(
                jnp.int32(0), e - t, jnp.int32(1), unroll=4,
                carry=tuple(jnp.zeros((16,), jnp.float32)
                            for _ in range(nreg)))(tok)
            lu = u - u0
            for c0 in range(nreg):
                plsc.addupdate(acc_v.at[lu, pl.ds(c0 * 16, 16)], regs[c0])
            return (u, e)

        return lax.while_loop(lambda st: st[1] < hi, ubody, st_in)

    def cbase(i):
        return jnp.minimum(t0a + i * _K, _TOTAL - _K)

    def ids_start(i, ids_b, isem_b):
        pltpu.async_copy(ids_hbm.at[pl.ds(cbase(i), _K)], ids_b, isem_b)

    def ids_wait(i, ids_b, isem_b):
        pltpu.make_async_copy(
            ids_hbm.at[pl.ds(cbase(i), _K)], ids_b, isem_b).wait()

    pltpu.sync_copy(ids_hbm.at[pl.ds(cbase(0), _K)], ids0)
    pltpu.async_copy(t1_hbm.at[ids0], rows0, gsem0)
    ids_start(1, ids1, isem1)

    def pair(p, st):
        ia = 2 * p + 1
        ids_wait(ia, ids1, isem1)
        pltpu.async_copy(t1_hbm.at[ids1], rows1, gsem1)
        pltpu.make_async_copy(t1_hbm.at[ids0], rows0, gsem0).wait()
        ids_start(ia + 1, ids0, isem0)
        st = walk(st, cbase(ia - 1), rows0)
        ids_wait(ia + 1, ids0, isem0)
        pltpu.async_copy(t1_hbm.at[ids0], rows0, gsem0)
        pltpu.make_async_copy(t1_hbm.at[ids1], rows1, gsem1).wait()
        ids_start(ia + 2, ids1, isem1)
        st = walk(st, cbase(ia), rows1)
        return st

    st_fin = lax.fori_loop(0, npairs, pair, (u0, t0), unroll=False)
    pltpu.make_async_copy(t1_hbm.at[ids0], rows0, gsem0).wait()
    ids_wait(2 * npairs + 1, ids1, isem1)
    return st_fin


def _seg_body(ids_hbm, cu_hbm, t1a_hbm, t1b_hbm, outa_hbm, outb_hbm,
              cu_v, ids0, ids1, rows0, rows1, acc_v,
              gsem0, gsem1, isem0, isem1):
    c = lax.axis_index("c")
    s = lax.axis_index("s")
    row0 = s * _UPT

    pltpu.sync_copy(cu_hbm, cu_v.at[pl.ds(0, _B + 1)])

    def zrow(r, cc):
        for c0 in range(_HH // 16):
            acc_v[r, pl.ds(c0 * 16, 16)] = jnp.zeros((16,), jnp.float32)
        return cc

    lax.fori_loop(0, _UPT, zrow, 0, unroll=False)

    @pl.when(c == 0)
    def _():
        _chunk_loop(s, ids_hbm, cu_v, t1a_hbm, ids0, ids1, rows0, rows1,
                    acc_v, gsem0, gsem1, isem0, isem1)
        pltpu.sync_copy(acc_v, outa_hbm.at[pl.ds(row0, _UPT)])

    @pl.when(c == 1)
    def _():
        _chunk_loop(s, ids_hbm, cu_v, t1b_hbm, ids0, ids1, rows0, rows1,
                    acc_v, gsem0, gsem1, isem0, isem1)
        pltpu.sync_copy(acc_v, outb_hbm.at[pl.ds(row0, _UPT)])


def _segsum(ids_padded, cu_pad, t1a, t1b):
    return pl.kernel(
        _seg_body,
        out_type=(
            jax.ShapeDtypeStruct((_B, _HH), jnp.float32),
            jax.ShapeDtypeStruct((_B, _HH), jnp.float32),
        ),
        mesh=plsc.VectorSubcoreMesh(
            core_axis_name="c", subcore_axis_name="s",
            num_cores=_NC, num_subcores=_NS,
        ),
        scratch_types=[
            pltpu.VMEM((_CUPAD,), jnp.int32),
            pltpu.VMEM((_K,), jnp.int32),
            pltpu.VMEM((_K,), jnp.int32),
            pltpu.VMEM((_K, _HW), jnp.float32),
            pltpu.VMEM((_K, _HW), jnp.float32),
            pltpu.VMEM((_UPT, _HH), jnp.float32),
            pltpu.SemaphoreType.DMA,
            pltpu.SemaphoreType.DMA,
            pltpu.SemaphoreType.DMA,
            pltpu.SemaphoreType.DMA,
        ],
        compiler_params=pltpu.CompilerParams(needs_layout_passes=False),
    )(ids_padded, cu_pad, t1a, t1b)


def _out_body(a_ref, b_ref, lo_ref, hi_ref, w2a_ref, w2b_ref, b2_ref, o_ref):
    cnt = (hi_ref[...] - lo_ref[...]).astype(jnp.float32)
    inv = 1.0 / jnp.maximum(cnt, 1.0)
    y = jnp.dot(a_ref[...] * inv, w2a_ref[...],
                preferred_element_type=jnp.float32)
    y = y + jnp.dot(b_ref[...] * inv, w2b_ref[...],
                    preferred_element_type=jnp.float32)
    y = y + b2_ref[...]
    o_ref[...] = jnp.where(cnt > 0.0, y, jnp.zeros_like(y))


def _finish(suma, sumb, cu_lo, cu_hi, w2a, w2b, b2_2d):
    return pl.pallas_call(
        _out_body,
        grid=(_B // _UB,),
        in_specs=[
            pl.BlockSpec((_UB, _HH), lambda i: (i, 0)),
            pl.BlockSpec((_UB, _HH), lambda i: (i, 0)),
            pl.BlockSpec((_UB, 1), lambda i: (i, 0)),
            pl.BlockSpec((_UB, 1), lambda i: (i, 0)),
            pl.BlockSpec((_HH, _D), lambda i: (0, 0)),
            pl.BlockSpec((_HH, _D), lambda i: (0, 0)),
            pl.BlockSpec((1, _D), lambda i: (0, 0)),
        ],
        out_specs=pl.BlockSpec((_UB, _D), lambda i: (i, 0)),
        out_shape=jax.ShapeDtypeStruct((_B, _D), jnp.float32),
    )(suma, sumb, cu_lo, cu_hi, w2a, w2b, b2_2d)


def kernel(flat_movie_ids, cu_seqlens, table, W1, b1, W2, b2):
    t1a, t1b = _mlp1(table, W1, b1.reshape(1, _H))
    suma, sumb = _segsum(flat_movie_ids, cu_seqlens, t1a, t1b)
    out = _finish(
        suma, sumb,
        cu_seqlens[:-1].reshape(_B, 1), cu_seqlens[1:].reshape(_B, 1),
        W2[:_HH], W2[_HH:], b2.reshape(1, _D),
    )
    return out
